# Initial kernel scaffold; baseline (speedup 1.0000x reference)
import jax
import jax.numpy as jnp
from jax.experimental import pallas as pl


def kernel(x, batch, W, b):
    raise NotImplementedError("write your pallas kernel here")


# speedup vs baseline: 6.7689x; 6.7689x over previous
import functools

import jax
import jax.numpy as jnp
from jax import lax
from jax.experimental import pallas as pl
from jax.experimental.pallas import tpu as pltpu
from jax.experimental.pallas import tpu_sc as plsc

N_NODES_K = 50000
D = 256
N_SEG = 512

N_TILES = 16
LANE = 128
CHUNKS = 25
PER_TILE = CHUNKS * LANE
N_PAD = N_TILES * PER_TILE
ACC = N_SEG + 16

TC_BLK = 2000


def _matvec_body(x_ref, w_ref, y_ref):
    y_ref[...] = jnp.dot(x_ref[...], w_ref[...],
                         preferred_element_type=jnp.float32)


def _matvec(x, W):
    return pl.pallas_call(
        _matvec_body,
        grid=(N_NODES_K // TC_BLK,),
        in_specs=[
            pl.BlockSpec((TC_BLK, D), lambda i: (i, 0)),
            pl.BlockSpec((D, 1), lambda i: (0, 0)),
        ],
        out_specs=pl.BlockSpec((TC_BLK, 1), lambda i: (i, 0)),
        out_shape=jax.ShapeDtypeStruct((N_NODES_K, 1), jnp.float32),
    )(x, W)


_MESH = plsc.VectorSubcoreMesh(
    core_axis_name="c", subcore_axis_name="s", num_cores=1
)


@functools.partial(
    pl.kernel,
    out_type=jax.ShapeDtypeStruct((N_SEG,), jnp.float32),
    mesh=_MESH,
    scratch_types=[
        pltpu.VMEM((CHUNKS, LANE), jnp.int32),
        pltpu.VMEM((CHUNKS, LANE), jnp.float32),
        pltpu.VMEM((LANE,), jnp.float32),
        pltpu.VMEM((2 * ACC,), jnp.float32),
        pltpu.VMEM_SHARED((ACC,), jnp.float32),
        pltpu.VMEM_SHARED((ACC,), jnp.float32),
        pltpu.VMEM((32,), jnp.float32),
        pltpu.VMEM((32,), jnp.float32),
        pltpu.VMEM((32,), jnp.float32),
        pltpu.VMEM((16,), jnp.float32),
    ],
)
def _sc_segment(y_hbm, ids_hbm, b_hbm, out_hbm,
                idx_v, y_v, ones_v, zeros_v, sums_sh, counts_sh,
                s_v, c_v, o_v, b_v):
    sid = lax.axis_index("s")

    pltpu.sync_copy(ids_hbm.at[sid], idx_v)
    pltpu.sync_copy(y_hbm.at[sid], y_v)
    pltpu.sync_copy(b_hbm, b_v)

    for i in range(LANE // 16):
        ones_v[pl.ds(i * 16, 16)] = jnp.ones((16,), jnp.float32)

    @pl.when(sid == 0)
    def _():
        for i in range(2 * ACC // 16):
            zeros_v[pl.ds(i * 16, 16)] = jnp.zeros((16,), jnp.float32)
        pltpu.sync_copy(zeros_v.at[pl.ds(0, ACC)], sums_sh)
        pltpu.sync_copy(zeros_v.at[pl.ds(ACC, ACC)], counts_sh)

    plsc.subcore_barrier()

    def body(j, carry):
        pltpu.sync_copy(y_v.at[j], sums_sh.at[idx_v.at[j]], add=True)
        pltpu.sync_copy(ones_v, counts_sh.at[idx_v.at[j]], add=True)
        return carry

    lax.fori_loop(0, CHUNKS, body, 0)

    plsc.subcore_barrier()

    base = sid * 32
    pltpu.sync_copy(sums_sh.at[pl.ds(base, 32)], s_v)
    pltpu.sync_copy(counts_sh.at[pl.ds(base, 32)], c_v)
    bvec = b_v[...]
    for i in range(2):
        s = s_v[pl.ds(i * 16, 16)]
        c = c_v[pl.ds(i * 16, 16)]
        z = s / jnp.maximum(c, 1.0) + bvec
        o_v[pl.ds(i * 16, 16)] = 1.0 / (1.0 + jnp.exp(-z))
    pltpu.sync_copy(o_v, out_hbm.at[pl.ds(base, 32)])


def kernel(x, batch, W, b):
    y = _matvec(x, W).reshape(N_NODES_K)

    n_extra = N_PAD - N_NODES_K
    y_pad = jnp.concatenate([y, jnp.zeros((n_extra,), jnp.float32)])
    ids = batch.astype(jnp.int32)
    pad_ids = N_SEG + (jnp.arange(n_extra, dtype=jnp.int32) % 16)
    ids_pad = jnp.concatenate([ids, pad_ids])

    y3 = y_pad.reshape(N_TILES, CHUNKS, LANE)
    ids3 = ids_pad.reshape(N_TILES, CHUNKS, LANE)
    b16 = jnp.broadcast_to(b.astype(jnp.float32), (16,))

    out = _sc_segment(y3, ids3, b16)
    return out.reshape(N_SEG, 1)


# speedup vs baseline: 7.3470x; 1.0854x over previous
import functools

import jax
import jax.numpy as jnp
from jax import lax
from jax.experimental import pallas as pl
from jax.experimental.pallas import tpu as pltpu
from jax.experimental.pallas import tpu_sc as plsc

N_NODES_K = 50000
D = 256
N_SEG = 512

N_TILES = 16
LANE = 128
CHUNKS = 25
PER_TILE = CHUNKS * LANE
N_PAD = N_TILES * PER_TILE
ACC = N_SEG + 16

TC_BLK = 5000


def _matvec_body(x_ref, w_ref, y_ref):
    y_ref[...] = jnp.dot(x_ref[...], w_ref[...],
                         preferred_element_type=jnp.float32)


def _matvec(x, W):
    return pl.pallas_call(
        _matvec_body,
        grid=(N_NODES_K // TC_BLK,),
        in_specs=[
            pl.BlockSpec((TC_BLK, D), lambda i: (i, 0)),
            pl.BlockSpec((D, 1), lambda i: (0, 0)),
        ],
        out_specs=pl.BlockSpec((TC_BLK, 1), lambda i: (i, 0)),
        out_shape=jax.ShapeDtypeStruct((N_PAD, 1), jnp.float32),
    )(x, W)


_MESH = plsc.VectorSubcoreMesh(
    core_axis_name="c", subcore_axis_name="s", num_cores=1
)


@functools.partial(
    pl.kernel,
    out_type=jax.ShapeDtypeStruct((N_SEG,), jnp.float32),
    mesh=_MESH,
    scratch_types=[
        pltpu.VMEM((CHUNKS, LANE), jnp.int32),
        pltpu.VMEM((CHUNKS, LANE), jnp.float32),
        pltpu.VMEM((CHUNKS, LANE), jnp.float32),
        pltpu.VMEM((2 * ACC,), jnp.float32),
        pltpu.VMEM_SHARED((ACC,), jnp.float32),
        pltpu.VMEM_SHARED((ACC,), jnp.float32),
        pltpu.VMEM((32,), jnp.float32),
        pltpu.VMEM((32,), jnp.float32),
        pltpu.VMEM((32,), jnp.float32),
        pltpu.VMEM((16,), jnp.float32),
        pltpu.SemaphoreType.DMA,
        pltpu.SemaphoreType.DMA,
    ],
)
def _sc_segment(y_hbm, ids_hbm, b_hbm, out_hbm,
                idx_v, y_v, ones_v, zeros_v, sums_sh, counts_sh,
                s_v, c_v, o_v, b_v, sem_a, sem_b):
    sid = lax.axis_index("s")

    pltpu.sync_copy(ids_hbm.at[sid], idx_v)
    pltpu.sync_copy(y_hbm.at[sid], y_v)
    pltpu.sync_copy(b_hbm, b_v)

    for j in range(CHUNKS):
        for i in range(LANE // 16):
            ones_v[j, pl.ds(i * 16, 16)] = jnp.ones((16,), jnp.float32)

    @pl.when(sid == 0)
    def _():
        for i in range(2 * ACC // 16):
            zeros_v[pl.ds(i * 16, 16)] = jnp.zeros((16,), jnp.float32)
        pltpu.sync_copy(zeros_v.at[pl.ds(0, ACC)], sums_sh)
        pltpu.sync_copy(zeros_v.at[pl.ds(ACC, ACC)], counts_sh)

    plsc.subcore_barrier()

    def body(j, carry):
        c1 = pltpu.async_copy(y_v.at[j], sums_sh.at[idx_v.at[j]], sem_a,
                              add=True)
        c2 = pltpu.async_copy(ones_v.at[j], counts_sh.at[idx_v.at[j]], sem_b,
                              add=True)
        c1.wait()
        c2.wait()
        return carry

    lax.fori_loop(0, CHUNKS, body, 0)

    plsc.subcore_barrier()

    base = sid * 32
    pltpu.sync_copy(sums_sh.at[pl.ds(base, 32)], s_v)
    pltpu.sync_copy(counts_sh.at[pl.ds(base, 32)], c_v)
    bvec = b_v[...]
    for i in range(2):
        s = s_v[pl.ds(i * 16, 16)]
        c = c_v[pl.ds(i * 16, 16)]
        z = s / jnp.maximum(c, 1.0) + bvec
        o_v[pl.ds(i * 16, 16)] = 1.0 / (1.0 + jnp.exp(-z))
    pltpu.sync_copy(o_v, out_hbm.at[pl.ds(base, 32)])


def kernel(x, batch, W, b):
    y_pad = _matvec(x, W).reshape(N_PAD)

    n_extra = N_PAD - N_NODES_K
    ids = batch.astype(jnp.int32)
    pad_ids = N_SEG + (jnp.arange(n_extra, dtype=jnp.int32) % 16)
    ids_pad = jnp.concatenate([ids, pad_ids])

    y3 = y_pad.reshape(N_TILES, CHUNKS, LANE)
    ids3 = ids_pad.reshape(N_TILES, CHUNKS, LANE)
    b16 = jnp.broadcast_to(b.astype(jnp.float32), (16,))

    out = _sc_segment(y3, ids3, b16)
    return out.reshape(N_SEG, 1)


# speedup vs baseline: 8.7381x; 1.1893x over previous
import functools

import jax
import jax.numpy as jnp
from jax import lax
from jax.experimental import pallas as pl
from jax.experimental.pallas import tpu as pltpu
from jax.experimental.pallas import tpu_sc as plsc

N_NODES_K = 50000
D = 256
N_SEG = 512

N_TILES = 16
LANE = 128
CHUNKS = 25
PER_TILE = CHUNKS * LANE
N_PAD = N_TILES * PER_TILE
ACC = N_SEG + 16

TC_BLK = 2048
TC_OUT_SUB = TC_BLK // 128


def _matvec_body(x_ref, w_ref, y_ref):
    v = jnp.dot(x_ref[...], w_ref[...], preferred_element_type=jnp.float32)
    y_ref[...] = v.reshape(TC_OUT_SUB, 128)


def _matvec(x, W):
    return pl.pallas_call(
        _matvec_body,
        grid=(N_PAD // TC_BLK,),
        in_specs=[
            pl.BlockSpec((TC_BLK, D), lambda i: (i, 0)),
            pl.BlockSpec((D, 1), lambda i: (0, 0)),
        ],
        out_specs=pl.BlockSpec((TC_OUT_SUB, 128), lambda i: (i, 0)),
        out_shape=jax.ShapeDtypeStruct((N_PAD // 128, 128), jnp.float32),
    )(x, W)


_MESH = plsc.VectorSubcoreMesh(
    core_axis_name="c", subcore_axis_name="s", num_cores=1
)


@functools.partial(
    pl.kernel,
    out_type=jax.ShapeDtypeStruct((N_SEG,), jnp.float32),
    mesh=_MESH,
    scratch_types=[
        pltpu.VMEM((CHUNKS, LANE), jnp.int32),
        pltpu.VMEM((CHUNKS, LANE), jnp.float32),
        pltpu.VMEM((CHUNKS, LANE), jnp.float32),
        pltpu.VMEM((2 * ACC,), jnp.float32),
        pltpu.VMEM_SHARED((ACC,), jnp.float32),
        pltpu.VMEM_SHARED((ACC,), jnp.float32),
        pltpu.VMEM((32,), jnp.float32),
        pltpu.VMEM((32,), jnp.float32),
        pltpu.VMEM((32,), jnp.float32),
        pltpu.VMEM((16,), jnp.float32),
        pltpu.SemaphoreType.DMA,
        pltpu.SemaphoreType.DMA,
    ],
)
def _sc_segment(y_hbm, ids_hbm, b_hbm, out_hbm,
                idx_v, y_v, ones_v, zeros_v, sums_sh, counts_sh,
                s_v, c_v, o_v, b_v, sem_a, sem_b):
    sid = lax.axis_index("s")

    d_ids = pltpu.async_copy(ids_hbm.at[sid], idx_v, sem_a)
    d_y = pltpu.async_copy(y_hbm.at[sid], y_v, sem_b)
    d_b = pltpu.async_copy(b_hbm, b_v, sem_a)

    for j in range(CHUNKS):
        for i in range(LANE // 16):
            ones_v[j, pl.ds(i * 16, 16)] = jnp.ones((16,), jnp.float32)
    for i in range(2 * ACC // 16):
        zeros_v[pl.ds(i * 16, 16)] = jnp.zeros((16,), jnp.float32)

    @pl.when(sid == 0)
    def _():
        pltpu.sync_copy(zeros_v.at[pl.ds(0, ACC)], sums_sh)

    @pl.when(sid == 1)
    def _():
        pltpu.sync_copy(zeros_v.at[pl.ds(ACC, ACC)], counts_sh)

    d_ids.wait()
    d_y.wait()
    d_b.wait()
    plsc.subcore_barrier()

    def fire(j, carry):
        pltpu.async_copy(y_v.at[j], sums_sh.at[idx_v.at[j]], sem_a, add=True)
        pltpu.async_copy(ones_v.at[j], counts_sh.at[idx_v.at[j]], sem_b,
                         add=True)
        return carry

    lax.fori_loop(0, CHUNKS, fire, 0)

    def drain(j, carry):
        pltpu.make_async_copy(y_v.at[j], sums_sh.at[idx_v.at[j]],
                              sem_a).wait()
        pltpu.make_async_copy(ones_v.at[j], counts_sh.at[idx_v.at[j]],
                              sem_b).wait()
        return carry

    lax.fori_loop(0, CHUNKS, drain, 0)

    plsc.subcore_barrier()

    base = sid * 32
    pltpu.sync_copy(sums_sh.at[pl.ds(base, 32)], s_v)
    pltpu.sync_copy(counts_sh.at[pl.ds(base, 32)], c_v)
    bvec = b_v[...]
    for i in range(2):
        s = s_v[pl.ds(i * 16, 16)]
        c = c_v[pl.ds(i * 16, 16)]
        z = s / jnp.maximum(c, 1.0) + bvec
        o_v[pl.ds(i * 16, 16)] = 1.0 / (1.0 + jnp.exp(-z))
    pltpu.sync_copy(o_v, out_hbm.at[pl.ds(base, 32)])


def kernel(x, batch, W, b):
    y2 = _matvec(x, W)

    n_extra = N_PAD - N_NODES_K
    ids = batch.astype(jnp.int32)
    pad_ids = N_SEG + (jnp.arange(n_extra, dtype=jnp.int32) % 16)
    ids_pad = jnp.concatenate([ids, pad_ids])

    ids3 = ids_pad.reshape(N_TILES, CHUNKS, LANE)
    y3 = y2.reshape(N_TILES, CHUNKS, LANE)
    b16 = jnp.broadcast_to(b.astype(jnp.float32), (16,))

    out = _sc_segment(y3, ids3, b16)
    return out.reshape(N_SEG, 1)


# speedup vs baseline: 8.7556x; 1.0020x over previous
import functools

import jax
import jax.numpy as jnp
from jax import lax
from jax.experimental import pallas as pl
from jax.experimental.pallas import tpu as pltpu
from jax.experimental.pallas import tpu_sc as plsc

N_NODES_K = 50000
D = 256
N_SEG = 512

N_TILES = 16
LANE = 128
CHUNKS = 25
PER_TILE = CHUNKS * LANE
N_PAD = N_TILES * PER_TILE
ACC = N_SEG + 16

TC_BLK = 2048
TC_OUT_SUB = TC_BLK // 128


def _matvec_body(x_ref, w_ref, y_ref):
    v = jnp.dot(x_ref[...], w_ref[...], preferred_element_type=jnp.float32)
    y_ref[...] = v.reshape(TC_OUT_SUB, 128)


def _matvec(x, W):
    return pl.pallas_call(
        _matvec_body,
        grid=(N_PAD // TC_BLK,),
        in_specs=[
            pl.BlockSpec((TC_BLK, D), lambda i: (i, 0)),
            pl.BlockSpec((D, 1), lambda i: (0, 0)),
        ],
        out_specs=pl.BlockSpec((TC_OUT_SUB, 128), lambda i: (i, 0)),
        out_shape=jax.ShapeDtypeStruct((N_PAD // 128, 128), jnp.float32),
        compiler_params=pltpu.CompilerParams(
            dimension_semantics=("parallel",)
        ),
    )(x, W)


_MESH = plsc.VectorSubcoreMesh(
    core_axis_name="c", subcore_axis_name="s", num_cores=1
)


@functools.partial(
    pl.kernel,
    out_type=jax.ShapeDtypeStruct((N_SEG,), jnp.float32),
    mesh=_MESH,
    scratch_types=[
        pltpu.VMEM((CHUNKS, LANE), jnp.int32),
        pltpu.VMEM((CHUNKS, LANE), jnp.float32),
        pltpu.VMEM((CHUNKS, LANE), jnp.float32),
        pltpu.VMEM((2 * ACC,), jnp.float32),
        pltpu.VMEM_SHARED((ACC,), jnp.float32),
        pltpu.VMEM_SHARED((ACC,), jnp.float32),
        pltpu.VMEM((32,), jnp.float32),
        pltpu.VMEM((32,), jnp.float32),
        pltpu.VMEM((32,), jnp.float32),
        pltpu.VMEM((16,), jnp.float32),
        pltpu.SemaphoreType.DMA,
        pltpu.SemaphoreType.DMA,
    ],
)
def _sc_segment(y_hbm, ids_hbm, b_hbm, out_hbm,
                idx_v, y_v, ones_v, zeros_v, sums_sh, counts_sh,
                s_v, c_v, o_v, b_v, sem_a, sem_b):
    sid = lax.axis_index("s")

    d_ids = pltpu.async_copy(ids_hbm.at[sid], idx_v, sem_a)
    d_y = pltpu.async_copy(y_hbm.at[sid], y_v, sem_b)
    d_b = pltpu.async_copy(b_hbm, b_v, sem_a)

    for j in range(CHUNKS):
        for i in range(LANE // 16):
            ones_v[j, pl.ds(i * 16, 16)] = jnp.ones((16,), jnp.float32)
    for i in range(2 * ACC // 16):
        zeros_v[pl.ds(i * 16, 16)] = jnp.zeros((16,), jnp.float32)

    @pl.when(sid == 0)
    def _():
        pltpu.sync_copy(zeros_v.at[pl.ds(0, ACC)], sums_sh)

    @pl.when(sid == 1)
    def _():
        pltpu.sync_copy(zeros_v.at[pl.ds(ACC, ACC)], counts_sh)

    d_ids.wait()
    d_y.wait()
    d_b.wait()
    plsc.subcore_barrier()

    def fire(j, carry):
        pltpu.async_copy(y_v.at[j], sums_sh.at[idx_v.at[j]], sem_a, add=True)
        pltpu.async_copy(ones_v.at[j], counts_sh.at[idx_v.at[j]], sem_b,
                         add=True)
        return carry

    lax.fori_loop(0, CHUNKS, fire, 0)

    def drain(j, carry):
        pltpu.make_async_copy(y_v.at[j], sums_sh.at[idx_v.at[j]],
                              sem_a).wait()
        pltpu.make_async_copy(ones_v.at[j], counts_sh.at[idx_v.at[j]],
                              sem_b).wait()
        return carry

    lax.fori_loop(0, CHUNKS, drain, 0)

    plsc.subcore_barrier()

    base = sid * 32
    pltpu.sync_copy(sums_sh.at[pl.ds(base, 32)], s_v)
    pltpu.sync_copy(counts_sh.at[pl.ds(base, 32)], c_v)
    bvec = b_v[...]
    for i in range(2):
        s = s_v[pl.ds(i * 16, 16)]
        c = c_v[pl.ds(i * 16, 16)]
        z = s / jnp.maximum(c, 1.0) + bvec
        o_v[pl.ds(i * 16, 16)] = 1.0 / (1.0 + jnp.exp(-z))
    pltpu.sync_copy(o_v, out_hbm.at[pl.ds(base, 32)])


def kernel(x, batch, W, b):
    y2 = _matvec(x, W)

    n_extra = N_PAD - N_NODES_K
    ids = batch.astype(jnp.int32)
    pad_ids = N_SEG + (jnp.arange(n_extra, dtype=jnp.int32) % 16)
    ids_pad = jnp.concatenate([ids, pad_ids])

    ids3 = ids_pad.reshape(N_TILES, CHUNKS, LANE)
    y3 = y2.reshape(N_TILES, CHUNKS, LANE)
    b16 = jnp.broadcast_to(b.astype(jnp.float32), (16,))

    out = _sc_segment(y3, ids3, b16)
    return out.reshape(N_SEG, 1)


# speedup vs baseline: 10.5318x; 1.2029x over previous
import functools

import jax
import jax.numpy as jnp
from jax import lax
from jax.experimental import pallas as pl
from jax.experimental.pallas import tpu as pltpu
from jax.experimental.pallas import tpu_sc as plsc

N_NODES_K = 50000
D = 256
N_SEG = 512

N_TILES = 16
LANE = 128
CHUNKS = 25
PER_TILE = CHUNKS * LANE
N_PAD = N_TILES * PER_TILE
ACC = N_SEG + 16

TC_BLK = 5120
TC_OUT_SUB = TC_BLK // 128


def _matvec_body(x_ref, w_ref, y_ref):
    v = jnp.dot(x_ref[...], w_ref[...], preferred_element_type=jnp.float32)
    y_ref[...] = v.reshape(TC_OUT_SUB, 128)


def _matvec(x, W):
    return pl.pallas_call(
        _matvec_body,
        grid=(N_PAD // TC_BLK,),
        in_specs=[
            pl.BlockSpec((TC_BLK, D), lambda i: (i, 0)),
            pl.BlockSpec((D, 1), lambda i: (0, 0)),
        ],
        out_specs=pl.BlockSpec((TC_OUT_SUB, 128), lambda i: (i, 0)),
        out_shape=jax.ShapeDtypeStruct((N_PAD // 128, 128), jnp.float32),
        compiler_params=pltpu.CompilerParams(
            dimension_semantics=("parallel",)
        ),
    )(x, W)


_MESH = plsc.VectorSubcoreMesh(
    core_axis_name="c", subcore_axis_name="s", num_cores=1
)


@functools.partial(
    pl.kernel,
    out_type=jax.ShapeDtypeStruct((N_SEG,), jnp.float32),
    mesh=_MESH,
    scratch_types=[
        pltpu.VMEM((CHUNKS, LANE), jnp.int32),
        pltpu.VMEM((CHUNKS, LANE), jnp.float32),
        pltpu.VMEM((CHUNKS, LANE), jnp.float32),
        pltpu.VMEM((2 * ACC,), jnp.float32),
        pltpu.VMEM_SHARED((ACC,), jnp.float32),
        pltpu.VMEM_SHARED((ACC,), jnp.float32),
        pltpu.VMEM((32,), jnp.float32),
        pltpu.VMEM((32,), jnp.float32),
        pltpu.VMEM((32,), jnp.float32),
        pltpu.VMEM((16,), jnp.float32),
        pltpu.SemaphoreType.DMA,
        pltpu.SemaphoreType.DMA,
    ],
)
def _sc_segment(y_hbm, ids_hbm, b_hbm, out_hbm,
                idx_v, y_v, ones_v, zeros_v, sums_sh, counts_sh,
                s_v, c_v, o_v, b_v, sem_a, sem_b):
    sid = lax.axis_index("s")

    d_ids = pltpu.async_copy(ids_hbm.at[sid], idx_v, sem_a)
    d_y = pltpu.async_copy(y_hbm.at[sid], y_v, sem_b)
    d_b = pltpu.async_copy(b_hbm, b_v, sem_a)

    for j in range(CHUNKS):
        for i in range(LANE // 16):
            ones_v[j, pl.ds(i * 16, 16)] = jnp.ones((16,), jnp.float32)
    for i in range(2 * ACC // 16):
        zeros_v[pl.ds(i * 16, 16)] = jnp.zeros((16,), jnp.float32)

    @pl.when(sid == 0)
    def _():
        pltpu.sync_copy(zeros_v.at[pl.ds(0, ACC)], sums_sh)

    @pl.when(sid == 1)
    def _():
        pltpu.sync_copy(zeros_v.at[pl.ds(ACC, ACC)], counts_sh)

    d_ids.wait()
    d_y.wait()
    d_b.wait()
    plsc.subcore_barrier()

    def fire(j, carry):
        pltpu.async_copy(y_v.at[j], sums_sh.at[idx_v.at[j]], sem_a, add=True)
        pltpu.async_copy(ones_v.at[j], counts_sh.at[idx_v.at[j]], sem_b,
                         add=True)
        return carry

    lax.fori_loop(0, CHUNKS, fire, 0)

    def drain(j, carry):
        pltpu.make_async_copy(y_v.at[j], sums_sh.at[idx_v.at[j]],
                              sem_a).wait()
        pltpu.make_async_copy(ones_v.at[j], counts_sh.at[idx_v.at[j]],
                              sem_b).wait()
        return carry

    lax.fori_loop(0, CHUNKS, drain, 0)

    plsc.subcore_barrier()

    base = sid * 32
    pltpu.sync_copy(sums_sh.at[pl.ds(base, 32)], s_v)
    pltpu.sync_copy(counts_sh.at[pl.ds(base, 32)], c_v)
    bvec = b_v[...]
    for i in range(2):
        s = s_v[pl.ds(i * 16, 16)]
        c = c_v[pl.ds(i * 16, 16)]
        z = s / jnp.maximum(c, 1.0) + bvec
        o_v[pl.ds(i * 16, 16)] = 1.0 / (1.0 + jnp.exp(-z))
    pltpu.sync_copy(o_v, out_hbm.at[pl.ds(base, 32)])


def kernel(x, batch, W, b):
    y2 = _matvec(x, W)

    n_extra = N_PAD - N_NODES_K
    ids = batch.astype(jnp.int32)
    pad_ids = N_SEG + (jnp.arange(n_extra, dtype=jnp.int32) % 16)
    ids_pad = jnp.concatenate([ids, pad_ids])

    ids3 = ids_pad.reshape(N_TILES, CHUNKS, LANE)
    y3 = y2.reshape(N_TILES, CHUNKS, LANE)
    b16 = jnp.broadcast_to(b.astype(jnp.float32), (16,))

    out = _sc_segment(y3, ids3, b16)
    return out.reshape(N_SEG, 1)


# speedup vs baseline: 10.7942x; 1.0249x over previous
import functools

import jax
import jax.numpy as jnp
from jax import lax
from jax.experimental import pallas as pl
from jax.experimental.pallas import tpu as pltpu
from jax.experimental.pallas import tpu_sc as plsc

N_NODES_K = 50000
D = 256
N_SEG = 512

N_TILES = 16
LANE = 128
CHUNKS = 25
PER_TILE = CHUNKS * LANE
N_PAD = N_TILES * PER_TILE
ACC = N_SEG + 16

TC_BLK = 10240
TC_OUT_SUB = TC_BLK // 128


def _matvec_body(x_ref, w_ref, y_ref):
    v = jnp.dot(x_ref[...], w_ref[...], preferred_element_type=jnp.float32)
    y_ref[...] = v.reshape(TC_OUT_SUB, 128)


def _matvec(x, W):
    return pl.pallas_call(
        _matvec_body,
        grid=(N_PAD // TC_BLK,),
        in_specs=[
            pl.BlockSpec((TC_BLK, D), lambda i: (i, 0)),
            pl.BlockSpec((D, 1), lambda i: (0, 0)),
        ],
        out_specs=pl.BlockSpec((TC_OUT_SUB, 128), lambda i: (i, 0)),
        out_shape=jax.ShapeDtypeStruct((N_PAD // 128, 128), jnp.float32),
        compiler_params=pltpu.CompilerParams(
            dimension_semantics=("parallel",)
        ),
    )(x, W)


_MESH = plsc.VectorSubcoreMesh(
    core_axis_name="c", subcore_axis_name="s", num_cores=1
)


@functools.partial(
    pl.kernel,
    out_type=jax.ShapeDtypeStruct((N_SEG,), jnp.float32),
    mesh=_MESH,
    scratch_types=[
        pltpu.VMEM((CHUNKS, LANE), jnp.int32),
        pltpu.VMEM((CHUNKS, LANE), jnp.float32),
        pltpu.VMEM((CHUNKS, LANE), jnp.float32),
        pltpu.VMEM((2 * ACC,), jnp.float32),
        pltpu.VMEM_SHARED((ACC,), jnp.float32),
        pltpu.VMEM_SHARED((ACC,), jnp.float32),
        pltpu.VMEM((32,), jnp.float32),
        pltpu.VMEM((32,), jnp.float32),
        pltpu.VMEM((32,), jnp.float32),
        pltpu.VMEM((16,), jnp.float32),
        pltpu.SemaphoreType.DMA,
        pltpu.SemaphoreType.DMA,
    ],
)
def _sc_segment(y_hbm, ids_hbm, b_hbm, out_hbm,
                idx_v, y_v, ones_v, zeros_v, sums_sh, counts_sh,
                s_v, c_v, o_v, b_v, sem_a, sem_b):
    sid = lax.axis_index("s")

    d_ids = pltpu.async_copy(ids_hbm.at[sid], idx_v, sem_a)
    d_y = pltpu.async_copy(y_hbm.at[sid], y_v, sem_b)
    d_b = pltpu.async_copy(b_hbm, b_v, sem_a)

    for j in range(CHUNKS):
        for i in range(LANE // 16):
            ones_v[j, pl.ds(i * 16, 16)] = jnp.ones((16,), jnp.float32)
    for i in range(2 * ACC // 16):
        zeros_v[pl.ds(i * 16, 16)] = jnp.zeros((16,), jnp.float32)

    @pl.when(sid == 0)
    def _():
        pltpu.sync_copy(zeros_v.at[pl.ds(0, ACC)], sums_sh)

    @pl.when(sid == 1)
    def _():
        pltpu.sync_copy(zeros_v.at[pl.ds(ACC, ACC)], counts_sh)

    d_ids.wait()
    d_y.wait()
    d_b.wait()
    plsc.subcore_barrier()

    def fire(j, carry):
        pltpu.async_copy(y_v.at[j], sums_sh.at[idx_v.at[j]], sem_a, add=True)
        pltpu.async_copy(ones_v.at[j], counts_sh.at[idx_v.at[j]], sem_b,
                         add=True)
        return carry

    lax.fori_loop(0, CHUNKS, fire, 0)

    def drain(j, carry):
        pltpu.make_async_copy(y_v.at[j], sums_sh.at[idx_v.at[j]],
                              sem_a).wait()
        pltpu.make_async_copy(ones_v.at[j], counts_sh.at[idx_v.at[j]],
                              sem_b).wait()
        return carry

    lax.fori_loop(0, CHUNKS, drain, 0)

    plsc.subcore_barrier()

    base = sid * 32
    pltpu.sync_copy(sums_sh.at[pl.ds(base, 32)], s_v)
    pltpu.sync_copy(counts_sh.at[pl.ds(base, 32)], c_v)
    bvec = b_v[...]
    for i in range(2):
        s = s_v[pl.ds(i * 16, 16)]
        c = c_v[pl.ds(i * 16, 16)]
        z = s / jnp.maximum(c, 1.0) + bvec
        o_v[pl.ds(i * 16, 16)] = 1.0 / (1.0 + jnp.exp(-z))
    pltpu.sync_copy(o_v, out_hbm.at[pl.ds(base, 32)])


def kernel(x, batch, W, b):
    y2 = _matvec(x, W)

    n_extra = N_PAD - N_NODES_K
    ids = batch.astype(jnp.int32)
    pad_ids = N_SEG + (jnp.arange(n_extra, dtype=jnp.int32) % 16)
    ids_pad = jnp.concatenate([ids, pad_ids])

    ids3 = ids_pad.reshape(N_TILES, CHUNKS, LANE)
    y3 = y2.reshape(N_TILES, CHUNKS, LANE)
    b16 = jnp.broadcast_to(b.astype(jnp.float32), (16,))

    out = _sc_segment(y3, ids3, b16)
    return out.reshape(N_SEG, 1)
